# baseline (speedup 1.0000x reference)
import functools

import jax
import jax.numpy as jnp
from jax import lax
from jax.experimental import pallas as pl
from jax.experimental.pallas import tpu as pltpu
from jax.experimental.pallas import tpu_sc as plsc

VOCAB = 1000
BATCH = 4096
HIST = 20
HIST_PAD = 24
LANES = 16

_info = plsc.get_sparse_core_info()
NUM_CORES = _info.num_cores
NUM_SUBCORES = _info.num_subcores
NUM_WORKERS = NUM_CORES * NUM_SUBCORES

B_BLOCK = BATCH // NUM_WORKERS
V_OCTETS = VOCAB // 8
VP_STEPS = (V_OCTETS + 1) // 2


def _make_tgather():
    mesh = plsc.VectorSubcoreMesh(core_axis_name="c", subcore_axis_name="s")

    @functools.partial(
        pl.kernel,
        mesh=mesh,
        out_type=jax.ShapeDtypeStruct((HIST, VOCAB, BATCH), jnp.float32),
        scratch_types=[
            pltpu.VMEM((HIST_PAD, B_BLOCK), jnp.int32),
            pltpu.VMEM((8 * VOCAB,), jnp.float32),
            pltpu.VMEM((8 * VOCAB,), jnp.float32),
            pltpu.VMEM((10, 8, B_BLOCK), jnp.float32),
            pltpu.SemaphoreType.DMA,
            pltpu.SemaphoreType.DMA,
        ] + [pltpu.SemaphoreType.DMA] * 10,
        compiler_params=pltpu.CompilerParams(needs_layout_passes=False),
    )
    def tgather_kernel(idx_hbm, tab_hbm, out_hbm, idx_v, st0, st1,
                       obr, sem_t0, sem_t1, *sem_o):
        cid = lax.axis_index("c")
        sid = lax.axis_index("s")
        wid = sid * NUM_CORES + cid
        bcol = wid * B_BLOCK

        pltpu.sync_copy(idx_hbm.at[:, pl.ds(bcol, B_BLOCK)], idx_v)

        def start_stage(vo, st, sem):
            pltpu.async_copy(tab_hbm.at[pl.ds(vo * 8 * VOCAB, 8 * VOCAB)],
                             st, sem)

        def wait_stage(vo, st, sem):
            pltpu.make_async_copy(
                tab_hbm.at[pl.ds(vo * 8 * VOCAB, 8 * VOCAB)], st, sem).wait()

        def out_dst(h, vo):
            return out_hbm.at[h, pl.ds(vo * 8, 8), pl.ds(bcol, B_BLOCK)]

        def fill_pair(vo, h0, st, bank, is_first):
            h1 = h0 + 1
            oba = obr.at[2 * bank]
            obb = obr.at[2 * bank + 1]
            sema = sem_o[2 * bank]
            semb = sem_o[2 * bank + 1]

            @pl.when(jnp.logical_not(is_first))
            def _():
                pltpu.make_async_copy(oba, out_dst(h0, vo), sema).wait()
                pltpu.make_async_copy(obb, out_dst(h1, vo), semb).wait()

            @plsc.parallel_loop(0, B_BLOCK // LANES, unroll=B_BLOCK // LANES)
            def _(lb):
                sl = pl.ds(lb * LANES, LANES)
                iv0 = idx_v[h0, sl]
                iv1 = idx_v[h1, sl]
                for s in range(8):
                    v0 = plsc.load_gather(st, [iv0 + (s * VOCAB)])
                    v1 = plsc.load_gather(st, [iv1 + (s * VOCAB)])
                    oba[s, sl] = v0
                    obb[s, sl] = v1
            pltpu.async_copy(oba, out_dst(h0, vo), sema)
            pltpu.async_copy(obb, out_dst(h1, vo), semb)

        def compute_octet(vo, st, is_first_octet):
            def hbody(hb, carry):
                first = jnp.logical_and(is_first_octet, hb == 0)
                base = 10 * hb
                for k in range(5):
                    fill_pair(vo, base + 2 * k, st, k, first)
                return carry
            lax.fori_loop(0, HIST // 10, hbody, 0)

        start_stage(0, st0, sem_t0)

        def vbody(vp, carry):
            v0 = 2 * vp
            v1 = v0 + 1

            @pl.when(v1 < V_OCTETS)
            def _():
                start_stage(v1, st1, sem_t1)

            wait_stage(v0, st0, sem_t0)
            compute_octet(v0, st0, v0 == 0)

            @pl.when(v1 < V_OCTETS)
            def _():
                @pl.when(v0 + 2 < V_OCTETS)
                def _():
                    start_stage(v0 + 2, st0, sem_t0)

                wait_stage(v1, st1, sem_t1)
                compute_octet(v1, st1, False)

            return carry

        lax.fori_loop(0, VP_STEPS, vbody, 0)

        for k in range(5):
            pltpu.make_async_copy(obr.at[2 * k],
                                  out_dst(10 + 2 * k, V_OCTETS - 1),
                                  sem_o[2 * k]).wait()
            pltpu.make_async_copy(obr.at[2 * k + 1],
                                  out_dst(11 + 2 * k, V_OCTETS - 1),
                                  sem_o[2 * k + 1]).wait()

    return tgather_kernel


_tgather = _make_tgather()


def kernel(idx, logits_table):
    idx_tp = jnp.pad(idx.T.astype(jnp.int32),
                     ((0, HIST_PAD - HIST), (0, 0)))
    tab_t = logits_table.T.reshape(-1)
    out_t = _tgather(idx_tp, tab_t)
    return jnp.transpose(out_t, (2, 0, 1))
